# baseline (speedup 1.0000x reference)
import functools

import jax
import jax.numpy as jnp
from jax import lax
from jax.experimental import pallas as pl
from jax.experimental.pallas import tpu as pltpu
from jax.experimental.pallas import tpu_sc as plsc

BATCH = 16384
CAT_DIM = 256
USER_DIM = 512
NC = 2
NS = 16
NW = NC * NS
ROWS_PER_W = BATCH // NW
CHUNK = 128
NCHUNK = ROWS_PER_W // CHUNK
L = 16


def _body(cat_ids, user_ids, lat, lon, wd, bd, emb_cat, emb_user,
          out,
          idx_cat_v, idx_user_v, lat_v, lon_v, cat_rows, user_rows, out_v,
          w_v, b_v, sem_c, sem_u):
    wid = lax.axis_index("s") * NC + lax.axis_index("c")
    base = wid * ROWS_PER_W
    pltpu.sync_copy(wd, w_v)
    pltpu.sync_copy(bd, b_v)
    lanes = lax.iota(jnp.int32, L)

    for c in range(NCHUNK):
        off = base + c * CHUNK
        pltpu.sync_copy(cat_ids.at[pl.ds(off, CHUNK)], idx_cat_v)
        pltpu.sync_copy(user_ids.at[pl.ds(off, CHUNK)], idx_user_v)
        pltpu.sync_copy(lat.at[pl.ds(off, CHUNK)], lat_v)
        pltpu.sync_copy(lon.at[pl.ds(off, CHUNK)], lon_v)
        cp1 = pltpu.async_copy(emb_cat.at[idx_cat_v], cat_rows, sem_c)
        cp2 = pltpu.async_copy(emb_user.at[idx_user_v], user_rows, sem_u)
        cp1.wait()
        cp2.wait()

        def gbody(g, _):
            rows = lanes + g * L
            lat16 = lat_v[pl.ds(g * L, L)]
            lon16 = lon_v[pl.ds(g * L, L)]

            def jobody(jo, carry):
                acc, s0, s1, sb = carry
                jbase = jo * L
                w0v = w_v[0, pl.ds(jbase, L)]
                w1v = w_v[1, pl.ds(jbase, L)]
                bv = b_v[pl.ds(jbase, L)]
                colb = jnp.full((L,), jbase, dtype=jnp.int32)
                for k in range(L):
                    col = colb + k
                    catc = plsc.load_gather(cat_rows, [rows, col])
                    u1c = plsc.load_gather(user_rows, [rows, col])
                    u2c = plsc.load_gather(user_rows, [rows, col + CAT_DIM])
                    acc = acc + catc * u1c
                    s0 = s0 + w0v[k] * u2c
                    s1 = s1 + w1v[k] * u2c
                    sb = sb + bv[k] * u2c
                return acc, s0, s1, sb

            z = jnp.zeros((L,), jnp.float32)
            acc, s0, s1, sb = lax.fori_loop(0, CAT_DIM // L, jobody,
                                            (z, z, z, z))
            out_v[pl.ds(g * L, L)] = acc + lat16 * s0 + lon16 * s1 + sb
            return 0

        lax.fori_loop(0, CHUNK // L, gbody, 0)
        pltpu.sync_copy(out_v, out.at[pl.ds(off, CHUNK)])


def kernel(category_ids, poi_lat, poi_lon, user_ids, W_dense, b_dense, emb_cat, emb_user):
    cat_ids = category_ids.reshape(BATCH).astype(jnp.int32)
    uids = user_ids.reshape(BATCH).astype(jnp.int32)
    lat = poi_lat.reshape(BATCH)
    lon = poi_lon.reshape(BATCH)
    mesh = plsc.VectorSubcoreMesh(core_axis_name="c", subcore_axis_name="s")
    f = pl.kernel(
        _body,
        out_type=jax.ShapeDtypeStruct((BATCH,), jnp.float32),
        mesh=mesh,
        compiler_params=pltpu.CompilerParams(
            use_tc_tiling_on_sc=False, needs_layout_passes=False),
        scratch_types=[
            pltpu.VMEM((CHUNK,), jnp.int32),
            pltpu.VMEM((CHUNK,), jnp.int32),
            pltpu.VMEM((CHUNK,), jnp.float32),
            pltpu.VMEM((CHUNK,), jnp.float32),
            pltpu.VMEM((CHUNK, CAT_DIM), jnp.float32),
            pltpu.VMEM((CHUNK, USER_DIM), jnp.float32),
            pltpu.VMEM((CHUNK,), jnp.float32),
            pltpu.VMEM((2, CAT_DIM), jnp.float32),
            pltpu.VMEM((CAT_DIM,), jnp.float32),
            pltpu.SemaphoreType.DMA,
            pltpu.SemaphoreType.DMA,
        ],
    )
    dot = f(cat_ids, uids, lat, lon, W_dense, b_dense, emb_cat, emb_user)
    return dot.reshape(BATCH, 1, 1)


# speedup vs baseline: 5.4114x; 5.4114x over previous
import functools

import jax
import jax.numpy as jnp
from jax import lax
from jax.experimental import pallas as pl
from jax.experimental.pallas import tpu as pltpu
from jax.experimental.pallas import tpu_sc as plsc

BATCH = 16384
CAT_DIM = 256
USER_DIM = 512
NC = 2
NS = 16
NW = NC * NS
ROWS_PER_W = BATCH // NW
CHUNK = 128
NCHUNK = ROWS_PER_W // CHUNK
L = 16
KT = CAT_DIM // L


def _body(cat_ids, user_ids, lat, lon, wd, bd, emb_cat, emb_user,
          out,
          idx_cat_v, idx_user_v, lat_v, lon_v, cat_rows, user_rows, out_v,
          w_v, b_v, sem_c, sem_u):
    wid = lax.axis_index("s") * NC + lax.axis_index("c")
    base = wid * ROWS_PER_W
    pltpu.sync_copy(wd, w_v)
    pltpu.sync_copy(bd, b_v)
    lanes = lax.iota(jnp.int32, L)

    def cbody(c, _c):
        off = base + c * CHUNK
        pltpu.sync_copy(cat_ids.at[pl.ds(off, CHUNK)], idx_cat_v)
        pltpu.sync_copy(user_ids.at[pl.ds(off, CHUNK)], idx_user_v)
        pltpu.sync_copy(lat.at[pl.ds(off, CHUNK)], lat_v)
        pltpu.sync_copy(lon.at[pl.ds(off, CHUNK)], lon_v)
        cp1 = pltpu.async_copy(emb_cat.at[idx_cat_v], cat_rows, sem_c)
        cp2 = pltpu.async_copy(emb_user.at[idx_user_v], user_rows, sem_u)
        cp1.wait()
        cp2.wait()

        def gbody(g, _g):
            rbase = g * L
            lat16 = lat_v[pl.ds(rbase, L)]
            lon16 = lon_v[pl.ds(rbase, L)]
            lats = [lat16[r] for r in range(L)]
            lons = [lon16[r] for r in range(L)]

            def kbody(k, accs):
                kf = k * L
                w0v = w_v[0, pl.ds(kf, L)]
                w1v = w_v[1, pl.ds(kf, L)]
                bv = b_v[pl.ds(kf, L)]
                new = []
                for r in range(L):
                    catv = cat_rows[rbase + r, pl.ds(kf, L)]
                    u1v = user_rows[rbase + r, pl.ds(kf, L)]
                    u2v = user_rows[rbase + r, pl.ds(kf + CAT_DIM, L)]
                    poi = lats[r] * w0v + lons[r] * w1v + bv
                    new.append(accs[r] + catv * u1v + poi * u2v)
                return tuple(new)

            z = jnp.zeros((L,), jnp.float32)
            accs = lax.fori_loop(0, KT, kbody, (z,) * L)
            res = z
            for r in range(L):
                s = jnp.sum(accs[r])
                res = jnp.where(lanes == r, s, res)
            out_v[pl.ds(rbase, L)] = res
            return _g

        lax.fori_loop(0, CHUNK // L, gbody, 0)
        pltpu.sync_copy(out_v, out.at[pl.ds(off, CHUNK)])
        return _c

    lax.fori_loop(0, NCHUNK, cbody, 0)


def kernel(category_ids, poi_lat, poi_lon, user_ids, W_dense, b_dense, emb_cat, emb_user):
    cat_ids = category_ids.reshape(BATCH).astype(jnp.int32)
    uids = user_ids.reshape(BATCH).astype(jnp.int32)
    lat = poi_lat.reshape(BATCH)
    lon = poi_lon.reshape(BATCH)
    mesh = plsc.VectorSubcoreMesh(core_axis_name="c", subcore_axis_name="s")
    f = pl.kernel(
        _body,
        out_type=jax.ShapeDtypeStruct((BATCH,), jnp.float32),
        mesh=mesh,
        compiler_params=pltpu.CompilerParams(needs_layout_passes=False),
        scratch_types=[
            pltpu.VMEM((CHUNK,), jnp.int32),
            pltpu.VMEM((CHUNK,), jnp.int32),
            pltpu.VMEM((CHUNK,), jnp.float32),
            pltpu.VMEM((CHUNK,), jnp.float32),
            pltpu.VMEM((CHUNK, CAT_DIM), jnp.float32),
            pltpu.VMEM((CHUNK, USER_DIM), jnp.float32),
            pltpu.VMEM((CHUNK,), jnp.float32),
            pltpu.VMEM((2, CAT_DIM), jnp.float32),
            pltpu.VMEM((CAT_DIM,), jnp.float32),
            pltpu.SemaphoreType.DMA,
            pltpu.SemaphoreType.DMA,
        ],
    )
    dot = f(cat_ids, uids, lat, lon, W_dense, b_dense, emb_cat, emb_user)
    return dot.reshape(BATCH, 1, 1)
